# Initial kernel scaffold; baseline (speedup 1.0000x reference)
import jax
import jax.numpy as jnp
from jax.experimental import pallas as pl


def kernel(x, edge_index, W1, b1, W2, b2):
    raise NotImplementedError("write your pallas kernel here")


# speedup vs baseline: 5.3820x; 5.3820x over previous
import functools

import jax
import jax.numpy as jnp
from jax import lax
from jax.experimental import pallas as pl
from jax.experimental.pallas import tpu as pltpu
from jax.experimental.pallas import tpu_sc as plsc

N = 10000
E = 320000
D = 128

NC = 2
NS = 16
NW = NC * NS

CHUNK = 80
EDGES_PER_TILE = E // NW
NUM_CHUNKS = EDGES_PER_TILE // CHUNK
NPAD = 10240
ROWS_PER_TILE = NPAD // NS
ZROWS = 80
ZREPS = ROWS_PER_TILE // ZROWS


def _sc_agg_body(src_hbm, dst_hbm, x_hbm, out_hbm,
                 src_v, dst_v, rows_v, zbuf, agg_sh, sem):
    cid = lax.axis_index("c")
    sid = lax.axis_index("s")
    wid = sid * NC + cid

    def zstore(i, _):
        r = i // 8
        c = (i % 8) * 16
        zbuf[r, pl.ds(c, 16)] = jnp.zeros((16,), jnp.float32)
        return 0
    lax.fori_loop(0, ZROWS * 8, zstore, 0)

    row0 = sid * ROWS_PER_TILE
    for k in range(ZREPS):
        pltpu.sync_copy(zbuf, agg_sh.at[pl.ds(row0 + k * ZROWS, ZROWS)])
    plsc.subcore_barrier()

    ebase = wid * EDGES_PER_TILE

    def body(i, _):
        off = ebase + i * CHUNK
        pltpu.sync_copy(src_hbm.at[pl.ds(off, CHUNK)], src_v)
        pltpu.sync_copy(dst_hbm.at[pl.ds(off, CHUNK)], dst_v)
        pltpu.async_copy(x_hbm.at[src_v], rows_v, sem).wait()
        pltpu.sync_copy(rows_v, agg_sh.at[dst_v], add=True)
        return 0
    lax.fori_loop(0, NUM_CHUNKS, body, 0)

    plsc.subcore_barrier()

    out_base = cid * NPAD + row0
    for k in range(ZREPS):
        pltpu.sync_copy(agg_sh.at[pl.ds(row0 + k * ZROWS, ZROWS)], zbuf)
        pltpu.sync_copy(zbuf, out_hbm.at[pl.ds(out_base + k * ZROWS, ZROWS)])


_sc_agg = functools.partial(
    pl.kernel,
    out_type=jax.ShapeDtypeStruct((NC * NPAD, D), jnp.float32),
    mesh=plsc.VectorSubcoreMesh(core_axis_name="c", subcore_axis_name="s"),
    scratch_types=[
        pltpu.VMEM((CHUNK,), jnp.int32),
        pltpu.VMEM((CHUNK,), jnp.int32),
        pltpu.VMEM((CHUNK, D), jnp.float32),
        pltpu.VMEM((ZROWS, D), jnp.float32),
        pltpu.VMEM_SHARED((NPAD, D), jnp.float32),
        pltpu.SemaphoreType.DMA,
    ],
)(_sc_agg_body)


ROWS_BLK = 1000


def _mlp_body(x_ref, a0_ref, a1_ref, w1_ref, b1_ref, w2_ref, b2_ref, o_ref):
    h = x_ref[...] + a0_ref[...] + a1_ref[...]
    h = lax.dot_general(h, w1_ref[...], (((1,), (1,)), ((), ())),
                        preferred_element_type=jnp.float32) + b1_ref[...]
    h = jnp.maximum(h, 0.0)
    h = lax.dot_general(h, w2_ref[...], (((1,), (1,)), ((), ())),
                        preferred_element_type=jnp.float32) + b2_ref[...]
    o_ref[...] = jnp.maximum(h, 0.0)


def _mlp(x, a0, a1, W1, b1, W2, b2):
    grid = (N // ROWS_BLK,)
    row_spec = pl.BlockSpec((ROWS_BLK, D), lambda i: (i, 0))
    full_spec = pl.BlockSpec((D, D), lambda i: (0, 0))
    bias_spec = pl.BlockSpec((D,), lambda i: (0,))
    return pl.pallas_call(
        _mlp_body,
        grid=grid,
        in_specs=[row_spec, row_spec, row_spec,
                  full_spec, bias_spec, full_spec, bias_spec],
        out_specs=row_spec,
        out_shape=jax.ShapeDtypeStruct((N, D), jnp.float32),
    )(x, a0, a1, W1, b1, W2, b2)


def kernel(x, edge_index, W1, b1, W2, b2):
    src = edge_index[0]
    dst = edge_index[1]
    aggs = _sc_agg(src, dst, x)
    return _mlp(x, aggs[:N], aggs[NPAD:NPAD + N], W1, b1, W2, b2)


# speedup vs baseline: 11.7640x; 2.1858x over previous
import functools

import jax
import jax.numpy as jnp
from jax import lax
from jax.experimental import pallas as pl
from jax.experimental.pallas import tpu as pltpu
from jax.experimental.pallas import tpu_sc as plsc

N = 10000
E = 320000
D = 128

NC = 2
NS = 16
NW = NC * NS

CHUNK = 80
EDGES_PER_TILE = E // NW
NUM_CHUNKS = EDGES_PER_TILE // CHUNK
RSLOTS = 4
ISLOTS = 6
GLEAD = 2
ILEAD = 4
NPAD = 10240
ROWS_PER_TILE = NPAD // NS
ZREPS = ROWS_PER_TILE // CHUNK


def _sc_agg_body(src_hbm, dst_hbm, x_hbm, out_hbm,
                 src_i, dst_i, rows_v, agg_sh, gsem, ssem, isem):
    cid = lax.axis_index("c")
    sid = lax.axis_index("s")
    wid = sid * NC + cid

    ebase = wid * EDGES_PER_TILE

    def fire_idx(j):
        s = j % ISLOTS
        off = ebase + j * CHUNK
        pltpu.async_copy(src_hbm.at[pl.ds(off, CHUNK)], src_i.at[s], isem.at[s])
        pltpu.async_copy(dst_hbm.at[pl.ds(off, CHUNK)], dst_i.at[s], isem.at[s])

    def wait_idx(j):
        s = j % ISLOTS
        off = ebase + j * CHUNK
        pltpu.make_async_copy(src_hbm.at[pl.ds(off, CHUNK)], src_i.at[s],
                              isem.at[s]).wait()
        pltpu.make_async_copy(dst_hbm.at[pl.ds(off, CHUNK)], dst_i.at[s],
                              isem.at[s]).wait()

    def fire_gather(j):
        pltpu.async_copy(x_hbm.at[src_i.at[j % ISLOTS]], rows_v.at[j % RSLOTS],
                         gsem.at[j % RSLOTS])

    def wait_gather(j):
        pltpu.make_async_copy(x_hbm.at[src_i.at[j % ISLOTS]],
                              rows_v.at[j % RSLOTS], gsem.at[j % RSLOTS]).wait()

    def fire_scatter(j):
        pltpu.async_copy(rows_v.at[j % RSLOTS], agg_sh.at[dst_i.at[j % ISLOTS]],
                         ssem.at[j % RSLOTS], add=True)

    def wait_scatter(j):
        pltpu.make_async_copy(rows_v.at[j % RSLOTS],
                              agg_sh.at[dst_i.at[j % ISLOTS]],
                              ssem.at[j % RSLOTS]).wait()

    def zstore(i, _):
        r = i // 8
        c = (i % 8) * 16
        rows_v[0, r, pl.ds(c, 16)] = jnp.zeros((16,), jnp.float32)
        return 0
    lax.fori_loop(0, CHUNK * 8, zstore, 0)

    row0 = sid * ROWS_PER_TILE
    for k in range(ZREPS):
        pltpu.sync_copy(rows_v.at[0], agg_sh.at[pl.ds(row0 + k * CHUNK, CHUNK)])

    for j in range(ILEAD):
        fire_idx(j)
    for j in range(GLEAD):
        wait_idx(j)
        fire_gather(j)

    plsc.subcore_barrier()

    def body(j, _):
        wait_gather(j)
        fire_scatter(j)

        @pl.when(j >= GLEAD)
        def _w():
            wait_scatter(j - GLEAD)

        @pl.when(j + ILEAD < NUM_CHUNKS)
        def _fi():
            fire_idx(j + ILEAD)

        @pl.when(j + GLEAD < NUM_CHUNKS)
        def _fg():
            wait_idx(j + GLEAD)
            fire_gather(j + GLEAD)
        return 0
    lax.fori_loop(0, NUM_CHUNKS, body, 0)

    for d in range(GLEAD):
        wait_scatter(NUM_CHUNKS - GLEAD + d)

    plsc.subcore_barrier()

    out_base = cid * NPAD + row0
    for k in range(ZREPS):
        pltpu.sync_copy(agg_sh.at[pl.ds(row0 + k * CHUNK, CHUNK)], rows_v.at[0])
        pltpu.sync_copy(rows_v.at[0], out_hbm.at[pl.ds(out_base + k * CHUNK, CHUNK)])


_sc_agg = functools.partial(
    pl.kernel,
    out_type=jax.ShapeDtypeStruct((NC * NPAD, D), jnp.float32),
    mesh=plsc.VectorSubcoreMesh(core_axis_name="c", subcore_axis_name="s"),
    scratch_types=[
        pltpu.VMEM((ISLOTS, CHUNK), jnp.int32),
        pltpu.VMEM((ISLOTS, CHUNK), jnp.int32),
        pltpu.VMEM((RSLOTS, CHUNK, D), jnp.float32),
        pltpu.VMEM_SHARED((NPAD, D), jnp.float32),
        pltpu.SemaphoreType.DMA((RSLOTS,)),
        pltpu.SemaphoreType.DMA((RSLOTS,)),
        pltpu.SemaphoreType.DMA((ISLOTS,)),
    ],
)(_sc_agg_body)


ROWS_BLK = 1000


def _mlp_body(x_ref, a0_ref, a1_ref, w1_ref, b1_ref, w2_ref, b2_ref, o_ref):
    h = x_ref[...] + a0_ref[...] + a1_ref[...]
    h = lax.dot_general(h, w1_ref[...], (((1,), (1,)), ((), ())),
                        preferred_element_type=jnp.float32) + b1_ref[...]
    h = jnp.maximum(h, 0.0)
    h = lax.dot_general(h, w2_ref[...], (((1,), (1,)), ((), ())),
                        preferred_element_type=jnp.float32) + b2_ref[...]
    o_ref[...] = jnp.maximum(h, 0.0)


def _mlp(x, a0, a1, W1, b1, W2, b2):
    grid = (N // ROWS_BLK,)
    row_spec = pl.BlockSpec((ROWS_BLK, D), lambda i: (i, 0))
    full_spec = pl.BlockSpec((D, D), lambda i: (0, 0))
    bias_spec = pl.BlockSpec((D,), lambda i: (0,))
    return pl.pallas_call(
        _mlp_body,
        grid=grid,
        in_specs=[row_spec, row_spec, row_spec,
                  full_spec, bias_spec, full_spec, bias_spec],
        out_specs=row_spec,
        out_shape=jax.ShapeDtypeStruct((N, D), jnp.float32),
    )(x, a0, a1, W1, b1, W2, b2)


def kernel(x, edge_index, W1, b1, W2, b2):
    src = edge_index[0]
    dst = edge_index[1]
    aggs = _sc_agg(src, dst, x)
    return _mlp(x, aggs[:N], aggs[NPAD:NPAD + N], W1, b1, W2, b2)


# speedup vs baseline: 12.2860x; 1.0444x over previous
import functools

import jax
import jax.numpy as jnp
from jax import lax
from jax.experimental import pallas as pl
from jax.experimental.pallas import tpu as pltpu
from jax.experimental.pallas import tpu_sc as plsc

N = 10000
E = 320000
D = 128

NC = 2
NS = 16
NW = NC * NS

CHUNK = 80
EDGES_PER_TILE = E // NW
NUM_CHUNKS = EDGES_PER_TILE // CHUNK
RSLOTS = 4
ISLOTS = 6
GLEAD = 2
ILEAD = 4
NPAD = 10240
ROWS_PER_TILE = NPAD // NS
ZREPS = ROWS_PER_TILE // CHUNK


def _sc_agg_body(src_hbm, dst_hbm, x_hbm, out0_hbm, out1_hbm,
                 src_i, dst_i, rows_v, agg_sh, gsem, ssem, isem):
    cid = lax.axis_index("c")
    sid = lax.axis_index("s")
    wid = sid * NC + cid

    ebase = wid * EDGES_PER_TILE

    def fire_idx(j):
        s = j % ISLOTS
        off = ebase + j * CHUNK
        pltpu.async_copy(src_hbm.at[pl.ds(off, CHUNK)], src_i.at[s], isem.at[s])
        pltpu.async_copy(dst_hbm.at[pl.ds(off, CHUNK)], dst_i.at[s], isem.at[s])

    def wait_idx(j):
        s = j % ISLOTS
        off = ebase + j * CHUNK
        pltpu.make_async_copy(src_hbm.at[pl.ds(off, CHUNK)], src_i.at[s],
                              isem.at[s]).wait()
        pltpu.make_async_copy(dst_hbm.at[pl.ds(off, CHUNK)], dst_i.at[s],
                              isem.at[s]).wait()

    def fire_gather(j):
        pltpu.async_copy(x_hbm.at[src_i.at[j % ISLOTS]], rows_v.at[j % RSLOTS],
                         gsem.at[j % RSLOTS])

    def wait_gather(j):
        pltpu.make_async_copy(x_hbm.at[src_i.at[j % ISLOTS]],
                              rows_v.at[j % RSLOTS], gsem.at[j % RSLOTS]).wait()

    def fire_scatter(j):
        pltpu.async_copy(rows_v.at[j % RSLOTS], agg_sh.at[dst_i.at[j % ISLOTS]],
                         ssem.at[j % RSLOTS], add=True)

    def wait_scatter(j):
        pltpu.make_async_copy(rows_v.at[j % RSLOTS],
                              agg_sh.at[dst_i.at[j % ISLOTS]],
                              ssem.at[j % RSLOTS]).wait()

    def zstore(i, _):
        r = i // 8
        c = (i % 8) * 16
        rows_v[0, r, pl.ds(c, 16)] = jnp.zeros((16,), jnp.float32)
        return 0
    lax.fori_loop(0, CHUNK * 8, zstore, 0)

    row0 = sid * ROWS_PER_TILE
    for k in range(ZREPS):
        pltpu.sync_copy(rows_v.at[0], agg_sh.at[pl.ds(row0 + k * CHUNK, CHUNK)])

    for j in range(ILEAD):
        fire_idx(j)
    for j in range(GLEAD):
        wait_idx(j)
        fire_gather(j)

    plsc.subcore_barrier()

    def body(j, _):
        wait_gather(j)
        fire_scatter(j)

        @pl.when(j >= GLEAD)
        def _w():
            wait_scatter(j - GLEAD)

        @pl.when(j + ILEAD < NUM_CHUNKS)
        def _fi():
            fire_idx(j + ILEAD)

        @pl.when(j + GLEAD < NUM_CHUNKS)
        def _fg():
            wait_idx(j + GLEAD)
            fire_gather(j + GLEAD)
        return 0
    lax.fori_loop(0, NUM_CHUNKS, body, 0)

    for d in range(GLEAD):
        wait_scatter(NUM_CHUNKS - GLEAD + d)

    plsc.subcore_barrier()

    for k in range(ZREPS):
        pltpu.sync_copy(agg_sh.at[pl.ds(row0 + k * CHUNK, CHUNK)], rows_v.at[0])

        @pl.when(cid == 0)
        def _c0():
            pltpu.sync_copy(rows_v.at[0],
                            out0_hbm.at[pl.ds(row0 + k * CHUNK, CHUNK)])

        @pl.when(cid == 1)
        def _c1():
            pltpu.sync_copy(rows_v.at[0],
                            out1_hbm.at[pl.ds(row0 + k * CHUNK, CHUNK)])


_sc_agg = functools.partial(
    pl.kernel,
    out_type=(jax.ShapeDtypeStruct((NPAD, D), jnp.float32),
              jax.ShapeDtypeStruct((NPAD, D), jnp.float32)),
    mesh=plsc.VectorSubcoreMesh(core_axis_name="c", subcore_axis_name="s"),
    scratch_types=[
        pltpu.VMEM((ISLOTS, CHUNK), jnp.int32),
        pltpu.VMEM((ISLOTS, CHUNK), jnp.int32),
        pltpu.VMEM((RSLOTS, CHUNK, D), jnp.float32),
        pltpu.VMEM_SHARED((NPAD, D), jnp.float32),
        pltpu.SemaphoreType.DMA((RSLOTS,)),
        pltpu.SemaphoreType.DMA((RSLOTS,)),
        pltpu.SemaphoreType.DMA((ISLOTS,)),
    ],
)(_sc_agg_body)


ROWS_BLK = 1000


def _mlp_body(x_ref, a0_ref, a1_ref, w1_ref, b1_ref, w2_ref, b2_ref, o_ref):
    h = x_ref[...] + a0_ref[...] + a1_ref[...]
    h = lax.dot_general(h, w1_ref[...], (((1,), (1,)), ((), ())),
                        preferred_element_type=jnp.float32) + b1_ref[...]
    h = jnp.maximum(h, 0.0)
    h = lax.dot_general(h, w2_ref[...], (((1,), (1,)), ((), ())),
                        preferred_element_type=jnp.float32) + b2_ref[...]
    o_ref[...] = jnp.maximum(h, 0.0)


def _mlp(x, a0, a1, W1, b1, W2, b2):
    grid = (N // ROWS_BLK,)
    row_spec = pl.BlockSpec((ROWS_BLK, D), lambda i: (i, 0))
    full_spec = pl.BlockSpec((D, D), lambda i: (0, 0))
    bias_spec = pl.BlockSpec((D,), lambda i: (0,))
    return pl.pallas_call(
        _mlp_body,
        grid=grid,
        in_specs=[row_spec, row_spec, row_spec,
                  full_spec, bias_spec, full_spec, bias_spec],
        out_specs=row_spec,
        out_shape=jax.ShapeDtypeStruct((N, D), jnp.float32),
    )(x, a0, a1, W1, b1, W2, b2)


def kernel(x, edge_index, W1, b1, W2, b2):
    src = edge_index[0]
    dst = edge_index[1]
    a0, a1 = _sc_agg(src, dst, x)
    return _mlp(x, a0, a1, W1, b1, W2, b2)


# speedup vs baseline: 12.5334x; 1.0201x over previous
import functools

import jax
import jax.numpy as jnp
from jax import lax
from jax.experimental import pallas as pl
from jax.experimental.pallas import tpu as pltpu
from jax.experimental.pallas import tpu_sc as plsc

N = 10000
E = 320000
D = 128

NC = 2
NS = 16
NW = NC * NS

CHUNK = 80
EDGES_PER_TILE = E // NW
NUM_CHUNKS = EDGES_PER_TILE // CHUNK
RSLOTS = 4
ISLOTS = 6
GLEAD = 2
ILEAD = 4
NPAD = 10240
ROWS_PER_TILE = NPAD // NS
ZREPS = ROWS_PER_TILE // CHUNK


def _sc_agg_body(src_hbm, dst_hbm, x_hbm, out0_hbm, out1_hbm,
                 src_i, dst_i, rows_v, agg_sh, gsem, ssem, isem):
    cid = lax.axis_index("c")
    sid = lax.axis_index("s")
    wid = sid * NC + cid

    ebase = wid * EDGES_PER_TILE

    def fire_idx(j):
        s = j % ISLOTS
        off = ebase + j * CHUNK
        pltpu.async_copy(src_hbm.at[pl.ds(off, CHUNK)], src_i.at[s], isem.at[s])
        pltpu.async_copy(dst_hbm.at[pl.ds(off, CHUNK)], dst_i.at[s], isem.at[s])

    def wait_idx(j):
        s = j % ISLOTS
        off = ebase + j * CHUNK
        pltpu.make_async_copy(src_hbm.at[pl.ds(off, CHUNK)], src_i.at[s],
                              isem.at[s]).wait()
        pltpu.make_async_copy(dst_hbm.at[pl.ds(off, CHUNK)], dst_i.at[s],
                              isem.at[s]).wait()

    def fire_gather(j):
        pltpu.async_copy(x_hbm.at[src_i.at[j % ISLOTS]], rows_v.at[j % RSLOTS],
                         gsem.at[j % RSLOTS])

    def wait_gather(j):
        pltpu.make_async_copy(x_hbm.at[src_i.at[j % ISLOTS]],
                              rows_v.at[j % RSLOTS], gsem.at[j % RSLOTS]).wait()

    def fire_scatter(j):
        pltpu.async_copy(rows_v.at[j % RSLOTS], agg_sh.at[dst_i.at[j % ISLOTS]],
                         ssem.at[j % RSLOTS], add=True)

    def wait_scatter(j):
        pltpu.make_async_copy(rows_v.at[j % RSLOTS],
                              agg_sh.at[dst_i.at[j % ISLOTS]],
                              ssem.at[j % RSLOTS]).wait()

    def zstore(i, _):
        r = i // 8
        c = (i % 8) * 16
        rows_v[0, r, pl.ds(c, 16)] = jnp.zeros((16,), jnp.float32)
        return 0
    lax.fori_loop(0, CHUNK * 8, zstore, 0)

    row0 = sid * ROWS_PER_TILE
    for k in range(ZREPS):
        pltpu.sync_copy(rows_v.at[0], agg_sh.at[pl.ds(row0 + k * CHUNK, CHUNK)])

    for j in range(ILEAD):
        fire_idx(j)
    for j in range(GLEAD):
        wait_idx(j)
        fire_gather(j)

    plsc.subcore_barrier()

    def body(j, _):
        wait_gather(j)
        fire_scatter(j)

        @pl.when(j >= GLEAD)
        def _w():
            wait_scatter(j - GLEAD)

        @pl.when(j + ILEAD < NUM_CHUNKS)
        def _fi():
            fire_idx(j + ILEAD)

        @pl.when(j + GLEAD < NUM_CHUNKS)
        def _fg():
            wait_idx(j + GLEAD)
            fire_gather(j + GLEAD)
        return 0
    lax.fori_loop(0, NUM_CHUNKS, body, 0)

    for d in range(GLEAD):
        wait_scatter(NUM_CHUNKS - GLEAD + d)

    plsc.subcore_barrier()

    for k in range(ZREPS):
        pltpu.sync_copy(agg_sh.at[pl.ds(row0 + k * CHUNK, CHUNK)], rows_v.at[0])

        @pl.when(cid == 0)
        def _c0():
            pltpu.sync_copy(rows_v.at[0],
                            out0_hbm.at[pl.ds(row0 + k * CHUNK, CHUNK)])

        @pl.when(cid == 1)
        def _c1():
            pltpu.sync_copy(rows_v.at[0],
                            out1_hbm.at[pl.ds(row0 + k * CHUNK, CHUNK)])


_sc_agg = functools.partial(
    pl.kernel,
    out_type=(jax.ShapeDtypeStruct((NPAD, D), jnp.float32),
              jax.ShapeDtypeStruct((NPAD, D), jnp.float32)),
    mesh=plsc.VectorSubcoreMesh(core_axis_name="c", subcore_axis_name="s"),
    scratch_types=[
        pltpu.VMEM((ISLOTS, CHUNK), jnp.int32),
        pltpu.VMEM((ISLOTS, CHUNK), jnp.int32),
        pltpu.VMEM((RSLOTS, CHUNK, D), jnp.float32),
        pltpu.VMEM_SHARED((NPAD, D), jnp.float32),
        pltpu.SemaphoreType.DMA((RSLOTS,)),
        pltpu.SemaphoreType.DMA((RSLOTS,)),
        pltpu.SemaphoreType.DMA((ISLOTS,)),
    ],
)(_sc_agg_body)


ROWS_BLK = 2000


def _mlp_body(x_ref, a0_ref, a1_ref, w1_ref, b1_ref, w2_ref, b2_ref, o_ref):
    h = x_ref[...] + a0_ref[...] + a1_ref[...]
    h = lax.dot_general(h, w1_ref[...], (((1,), (1,)), ((), ())),
                        preferred_element_type=jnp.float32) + b1_ref[...]
    h = jnp.maximum(h, 0.0)
    h = lax.dot_general(h, w2_ref[...], (((1,), (1,)), ((), ())),
                        preferred_element_type=jnp.float32) + b2_ref[...]
    o_ref[...] = jnp.maximum(h, 0.0)


def _mlp(x, a0, a1, W1, b1, W2, b2):
    grid = (N // ROWS_BLK,)
    row_spec = pl.BlockSpec((ROWS_BLK, D), lambda i: (i, 0))
    full_spec = pl.BlockSpec((D, D), lambda i: (0, 0))
    bias_spec = pl.BlockSpec((D,), lambda i: (0,))
    return pl.pallas_call(
        _mlp_body,
        grid=grid,
        in_specs=[row_spec, row_spec, row_spec,
                  full_spec, bias_spec, full_spec, bias_spec],
        out_specs=row_spec,
        out_shape=jax.ShapeDtypeStruct((N, D), jnp.float32),
    )(x, a0, a1, W1, b1, W2, b2)


def kernel(x, edge_index, W1, b1, W2, b2):
    src = edge_index[0]
    dst = edge_index[1]
    a0, a1 = _sc_agg(src, dst, x)
    return _mlp(x, a0, a1, W1, b1, W2, b2)


# speedup vs baseline: 13.1002x; 1.0452x over previous
import functools

import jax
import jax.numpy as jnp
from jax import lax
from jax.experimental import pallas as pl
from jax.experimental.pallas import tpu as pltpu
from jax.experimental.pallas import tpu_sc as plsc

N = 10000
E = 320000
D = 128

NC = 2
NS = 16
NW = NC * NS

CHUNK = 40
EDGES_PER_TILE = E // NW
NUM_CHUNKS = EDGES_PER_TILE // CHUNK
RSLOTS = 8
ISLOTS = 12
GLEAD = 4
ILEAD = 8
NPAD = 10240
ROWS_PER_TILE = NPAD // NS
ZREPS = ROWS_PER_TILE // CHUNK


def _sc_agg_body(src_hbm, dst_hbm, x_hbm, out0_hbm, out1_hbm,
                 src_i, dst_i, rows_v, agg_sh, gsem, ssem, isem):
    cid = lax.axis_index("c")
    sid = lax.axis_index("s")
    wid = sid * NC + cid

    ebase = wid * EDGES_PER_TILE

    def fire_idx(j):
        s = j % ISLOTS
        off = ebase + j * CHUNK
        pltpu.async_copy(src_hbm.at[pl.ds(off, CHUNK)], src_i.at[s], isem.at[s])
        pltpu.async_copy(dst_hbm.at[pl.ds(off, CHUNK)], dst_i.at[s], isem.at[s])

    def wait_idx(j):
        s = j % ISLOTS
        off = ebase + j * CHUNK
        pltpu.make_async_copy(src_hbm.at[pl.ds(off, CHUNK)], src_i.at[s],
                              isem.at[s]).wait()
        pltpu.make_async_copy(dst_hbm.at[pl.ds(off, CHUNK)], dst_i.at[s],
                              isem.at[s]).wait()

    def fire_gather(j):
        pltpu.async_copy(x_hbm.at[src_i.at[j % ISLOTS]], rows_v.at[j % RSLOTS],
                         gsem.at[j % RSLOTS])

    def wait_gather(j):
        pltpu.make_async_copy(x_hbm.at[src_i.at[j % ISLOTS]],
                              rows_v.at[j % RSLOTS], gsem.at[j % RSLOTS]).wait()

    def fire_scatter(j):
        pltpu.async_copy(rows_v.at[j % RSLOTS], agg_sh.at[dst_i.at[j % ISLOTS]],
                         ssem.at[j % RSLOTS], add=True)

    def wait_scatter(j):
        pltpu.make_async_copy(rows_v.at[j % RSLOTS],
                              agg_sh.at[dst_i.at[j % ISLOTS]],
                              ssem.at[j % RSLOTS]).wait()

    def zstore(i, _):
        r = i // 8
        c = (i % 8) * 16
        rows_v[0, r, pl.ds(c, 16)] = jnp.zeros((16,), jnp.float32)
        return 0
    lax.fori_loop(0, CHUNK * 8, zstore, 0)

    row0 = sid * ROWS_PER_TILE
    for k in range(ZREPS):
        pltpu.sync_copy(rows_v.at[0], agg_sh.at[pl.ds(row0 + k * CHUNK, CHUNK)])

    for j in range(ILEAD):
        fire_idx(j)
    for j in range(GLEAD):
        wait_idx(j)
        fire_gather(j)

    plsc.subcore_barrier()

    def body(j, _):
        wait_gather(j)
        fire_scatter(j)

        @pl.when(j >= GLEAD)
        def _w():
            wait_scatter(j - GLEAD)

        @pl.when(j + ILEAD < NUM_CHUNKS)
        def _fi():
            fire_idx(j + ILEAD)

        @pl.when(j + GLEAD < NUM_CHUNKS)
        def _fg():
            wait_idx(j + GLEAD)
            fire_gather(j + GLEAD)
        return 0
    lax.fori_loop(0, NUM_CHUNKS, body, 0)

    for d in range(GLEAD):
        wait_scatter(NUM_CHUNKS - GLEAD + d)

    plsc.subcore_barrier()

    for k in range(ZREPS):
        pltpu.sync_copy(agg_sh.at[pl.ds(row0 + k * CHUNK, CHUNK)], rows_v.at[0])

        @pl.when(cid == 0)
        def _c0():
            pltpu.sync_copy(rows_v.at[0],
                            out0_hbm.at[pl.ds(row0 + k * CHUNK, CHUNK)])

        @pl.when(cid == 1)
        def _c1():
            pltpu.sync_copy(rows_v.at[0],
                            out1_hbm.at[pl.ds(row0 + k * CHUNK, CHUNK)])


_sc_agg = functools.partial(
    pl.kernel,
    out_type=(jax.ShapeDtypeStruct((NPAD, D), jnp.float32),
              jax.ShapeDtypeStruct((NPAD, D), jnp.float32)),
    mesh=plsc.VectorSubcoreMesh(core_axis_name="c", subcore_axis_name="s"),
    scratch_types=[
        pltpu.VMEM((ISLOTS, CHUNK), jnp.int32),
        pltpu.VMEM((ISLOTS, CHUNK), jnp.int32),
        pltpu.VMEM((RSLOTS, CHUNK, D), jnp.float32),
        pltpu.VMEM_SHARED((NPAD, D), jnp.float32),
        pltpu.SemaphoreType.DMA((RSLOTS,)),
        pltpu.SemaphoreType.DMA((RSLOTS,)),
        pltpu.SemaphoreType.DMA((ISLOTS,)),
    ],
)(_sc_agg_body)


ROWS_BLK = 2000


def _mlp_body(x_ref, a0_ref, a1_ref, w1_ref, b1_ref, w2_ref, b2_ref, o_ref):
    h = x_ref[...] + a0_ref[...] + a1_ref[...]
    h = lax.dot_general(h, w1_ref[...], (((1,), (1,)), ((), ())),
                        preferred_element_type=jnp.float32) + b1_ref[...]
    h = jnp.maximum(h, 0.0)
    h = lax.dot_general(h, w2_ref[...], (((1,), (1,)), ((), ())),
                        preferred_element_type=jnp.float32) + b2_ref[...]
    o_ref[...] = jnp.maximum(h, 0.0)


def _mlp(x, a0, a1, W1, b1, W2, b2):
    grid = (N // ROWS_BLK,)
    row_spec = pl.BlockSpec((ROWS_BLK, D), lambda i: (i, 0))
    full_spec = pl.BlockSpec((D, D), lambda i: (0, 0))
    bias_spec = pl.BlockSpec((D,), lambda i: (0,))
    return pl.pallas_call(
        _mlp_body,
        grid=grid,
        in_specs=[row_spec, row_spec, row_spec,
                  full_spec, bias_spec, full_spec, bias_spec],
        out_specs=row_spec,
        out_shape=jax.ShapeDtypeStruct((N, D), jnp.float32),
    )(x, a0, a1, W1, b1, W2, b2)


def kernel(x, edge_index, W1, b1, W2, b2):
    src = edge_index[0]
    dst = edge_index[1]
    a0, a1 = _sc_agg(src, dst, x)
    return _mlp(x, a0, a1, W1, b1, W2, b2)


# speedup vs baseline: 13.1528x; 1.0040x over previous
import functools

import jax
import jax.numpy as jnp
from jax import lax
from jax.experimental import pallas as pl
from jax.experimental.pallas import tpu as pltpu
from jax.experimental.pallas import tpu_sc as plsc

N = 10000
E = 320000
D = 128

NC = 2
NS = 16
NW = NC * NS

CHUNK = 40
EDGES_PER_TILE = E // NW
NUM_CHUNKS = EDGES_PER_TILE // CHUNK
RSLOTS = 8
ISLOTS = 12
GLEAD = 4
ILEAD = 8
NPAD = 10240
ROWS_PER_TILE = NPAD // NS
ZREPS = ROWS_PER_TILE // CHUNK


def _sc_agg_body(src_hbm, dst_hbm, x_hbm, out0_hbm, out1_hbm,
                 src_i, dst_i, rows_v, agg_sh, gsem, ssem, isem,
                 zsem, psem, hsem):
    zbuf = rows_v.at[RSLOTS - 1]
    cid = lax.axis_index("c")
    sid = lax.axis_index("s")
    wid = sid * NC + cid

    ebase = wid * EDGES_PER_TILE

    def fire_idx(j):
        s = j % ISLOTS
        off = ebase + j * CHUNK
        pltpu.async_copy(src_hbm.at[pl.ds(off, CHUNK)], src_i.at[s], isem.at[s])
        pltpu.async_copy(dst_hbm.at[pl.ds(off, CHUNK)], dst_i.at[s], isem.at[s])

    def wait_idx(j):
        s = j % ISLOTS
        off = ebase + j * CHUNK
        pltpu.make_async_copy(src_hbm.at[pl.ds(off, CHUNK)], src_i.at[s],
                              isem.at[s]).wait()
        pltpu.make_async_copy(dst_hbm.at[pl.ds(off, CHUNK)], dst_i.at[s],
                              isem.at[s]).wait()

    def fire_gather(j):
        pltpu.async_copy(x_hbm.at[src_i.at[j % ISLOTS]], rows_v.at[j % RSLOTS],
                         gsem.at[j % RSLOTS])

    def wait_gather(j):
        pltpu.make_async_copy(x_hbm.at[src_i.at[j % ISLOTS]],
                              rows_v.at[j % RSLOTS], gsem.at[j % RSLOTS]).wait()

    def fire_scatter(j):
        pltpu.async_copy(rows_v.at[j % RSLOTS], agg_sh.at[dst_i.at[j % ISLOTS]],
                         ssem.at[j % RSLOTS], add=True)

    def wait_scatter(j):
        pltpu.make_async_copy(rows_v.at[j % RSLOTS],
                              agg_sh.at[dst_i.at[j % ISLOTS]],
                              ssem.at[j % RSLOTS]).wait()

    def zstore(i, _):
        r = i // 8
        c = (i % 8) * 16
        zbuf[r, pl.ds(c, 16)] = jnp.zeros((16,), jnp.float32)
        return 0
    lax.fori_loop(0, CHUNK * 8, zstore, 0)

    row0 = sid * ROWS_PER_TILE
    for j in range(ILEAD):
        fire_idx(j)
    for w in range(ZREPS // 4):
        for k in range(w * 4, w * 4 + 4):
            pltpu.async_copy(zbuf, agg_sh.at[pl.ds(row0 + k * CHUNK, CHUNK)],
                             zsem)
        for k in range(w * 4, w * 4 + 4):
            pltpu.make_async_copy(zbuf,
                                  agg_sh.at[pl.ds(row0 + k * CHUNK, CHUNK)],
                                  zsem).wait()
    for j in range(GLEAD):
        wait_idx(j)
        fire_gather(j)

    plsc.subcore_barrier()

    def body(j, _):
        wait_gather(j)
        fire_scatter(j)

        @pl.when(j >= GLEAD)
        def _w():
            wait_scatter(j - GLEAD)

        @pl.when(j + ILEAD < NUM_CHUNKS)
        def _fi():
            fire_idx(j + ILEAD)

        @pl.when(j + GLEAD < NUM_CHUNKS)
        def _fg():
            wait_idx(j + GLEAD)
            fire_gather(j + GLEAD)
        return 0
    lax.fori_loop(0, NUM_CHUNKS, body, 0)

    for d in range(GLEAD):
        wait_scatter(NUM_CHUNKS - GLEAD + d)

    plsc.subcore_barrier()

    for k in range(ZREPS):
        rows = pl.ds(row0 + k * CHUNK, CHUNK)
        pltpu.sync_copy(agg_sh.at[rows], rows_v.at[0])

        @pl.when(cid == 0)
        def _c0():
            pltpu.sync_copy(rows_v.at[0], out0_hbm.at[rows])

        @pl.when(cid == 1)
        def _c1():
            pltpu.sync_copy(rows_v.at[0], out1_hbm.at[rows])


_sc_agg = functools.partial(
    pl.kernel,
    out_type=(jax.ShapeDtypeStruct((NPAD, D), jnp.float32),
              jax.ShapeDtypeStruct((NPAD, D), jnp.float32)),
    mesh=plsc.VectorSubcoreMesh(core_axis_name="c", subcore_axis_name="s"),
    scratch_types=[
        pltpu.VMEM((ISLOTS, CHUNK), jnp.int32),
        pltpu.VMEM((ISLOTS, CHUNK), jnp.int32),
        pltpu.VMEM((RSLOTS, CHUNK, D), jnp.float32),
        pltpu.VMEM_SHARED((NPAD, D), jnp.float32),
        pltpu.SemaphoreType.DMA((RSLOTS,)),
        pltpu.SemaphoreType.DMA((RSLOTS,)),
        pltpu.SemaphoreType.DMA((ISLOTS,)),
        pltpu.SemaphoreType.DMA,
        pltpu.SemaphoreType.DMA((2,)),
        pltpu.SemaphoreType.DMA((2,)),
    ],
)(_sc_agg_body)


ROWS_BLK = 2000


def _mlp_body(x_ref, a0_ref, a1_ref, w1_ref, b1_ref, w2_ref, b2_ref, o_ref):
    h = x_ref[...] + a0_ref[...] + a1_ref[...]
    h = lax.dot_general(h, w1_ref[...], (((1,), (1,)), ((), ())),
                        preferred_element_type=jnp.float32) + b1_ref[...]
    h = jnp.maximum(h, 0.0)
    h = lax.dot_general(h, w2_ref[...], (((1,), (1,)), ((), ())),
                        preferred_element_type=jnp.float32) + b2_ref[...]
    o_ref[...] = jnp.maximum(h, 0.0)


def _mlp(x, a0, a1, W1, b1, W2, b2):
    grid = (N // ROWS_BLK,)
    row_spec = pl.BlockSpec((ROWS_BLK, D), lambda i: (i, 0))
    full_spec = pl.BlockSpec((D, D), lambda i: (0, 0))
    bias_spec = pl.BlockSpec((D,), lambda i: (0,))
    return pl.pallas_call(
        _mlp_body,
        grid=grid,
        in_specs=[row_spec, row_spec, row_spec,
                  full_spec, bias_spec, full_spec, bias_spec],
        out_specs=row_spec,
        out_shape=jax.ShapeDtypeStruct((N, D), jnp.float32),
    )(x, a0, a1, W1, b1, W2, b2)


def kernel(x, edge_index, W1, b1, W2, b2):
    src = edge_index[0]
    dst = edge_index[1]
    a0, a1 = _sc_agg(src, dst, x)
    return _mlp(x, a0, a1, W1, b1, W2, b2)


# speedup vs baseline: 13.3853x; 1.0177x over previous
import functools

import jax
import jax.numpy as jnp
from jax import lax
from jax.experimental import pallas as pl
from jax.experimental.pallas import tpu as pltpu
from jax.experimental.pallas import tpu_sc as plsc

N = 10000
E = 320000
D = 128

NC = 2
NS = 16
NW = NC * NS

CHUNK = 40
EDGES_PER_TILE = E // NW
NUM_CHUNKS = EDGES_PER_TILE // CHUNK
RSLOTS = 8
ISLOTS = 12
GLEAD = 4
ILEAD = 8
NPAD = 10240
ROWS_PER_TILE = NPAD // NS
ZREPS = ROWS_PER_TILE // CHUNK


def _sc_agg_body(src_hbm, dst_hbm, x_hbm, out0_hbm, out1_hbm,
                 src_i, dst_i, rows_v, agg_sh, gsem, ssem, isem,
                 zsem, psem, hsem):
    zbuf = rows_v.at[pl.ds((RSLOTS - 1) * CHUNK, CHUNK)]
    cid = lax.axis_index("c")
    sid = lax.axis_index("s")
    wid = sid * NC + cid

    ebase = wid * EDGES_PER_TILE

    def fire_idx(j):
        s = j % ISLOTS
        off = ebase + j * CHUNK
        pltpu.async_copy(src_hbm.at[pl.ds(off, CHUNK)], src_i.at[s], isem.at[s])
        pltpu.async_copy(dst_hbm.at[pl.ds(off, CHUNK)], dst_i.at[s], isem.at[s])

    def wait_idx(j):
        s = j % ISLOTS
        off = ebase + j * CHUNK
        pltpu.make_async_copy(src_hbm.at[pl.ds(off, CHUNK)], src_i.at[s],
                              isem.at[s]).wait()
        pltpu.make_async_copy(dst_hbm.at[pl.ds(off, CHUNK)], dst_i.at[s],
                              isem.at[s]).wait()

    def rslot(j):
        return rows_v.at[pl.ds((j % RSLOTS) * CHUNK, CHUNK)]

    def fire_gather(j):
        pltpu.async_copy(x_hbm.at[src_i.at[j % ISLOTS]], rslot(j),
                         gsem.at[j % RSLOTS])

    def wait_gather(j):
        pltpu.make_async_copy(x_hbm.at[src_i.at[j % ISLOTS]],
                              rslot(j), gsem.at[j % RSLOTS]).wait()

    def fire_scatter(j):
        pltpu.async_copy(rslot(j), agg_sh.at[dst_i.at[j % ISLOTS]],
                         ssem.at[j % RSLOTS], add=True)

    def wait_scatter(j):
        pltpu.make_async_copy(rslot(j),
                              agg_sh.at[dst_i.at[j % ISLOTS]],
                              ssem.at[j % RSLOTS]).wait()

    def zstore(i, _):
        r = i // 8
        c = (i % 8) * 16
        zbuf[r, pl.ds(c, 16)] = jnp.zeros((16,), jnp.float32)
        return 0
    lax.fori_loop(0, CHUNK * 8, zstore, 0)

    row0 = sid * ROWS_PER_TILE
    for j in range(ILEAD):
        fire_idx(j)
    for w in range(ZREPS // 4):
        for k in range(w * 4, w * 4 + 4):
            pltpu.async_copy(zbuf, agg_sh.at[pl.ds(row0 + k * CHUNK, CHUNK)],
                             zsem)
        for k in range(w * 4, w * 4 + 4):
            pltpu.make_async_copy(zbuf,
                                  agg_sh.at[pl.ds(row0 + k * CHUNK, CHUNK)],
                                  zsem).wait()
    for j in range(GLEAD):
        wait_idx(j)
        fire_gather(j)

    plsc.subcore_barrier()

    def body(j, _):
        wait_gather(j)
        fire_scatter(j)

        @pl.when(j >= GLEAD)
        def _w():
            wait_scatter(j - GLEAD)

        @pl.when(j + ILEAD < NUM_CHUNKS)
        def _fi():
            fire_idx(j + ILEAD)

        @pl.when(j + GLEAD < NUM_CHUNKS)
        def _fg():
            wait_idx(j + GLEAD)
            fire_gather(j + GLEAD)
        return 0
    lax.fori_loop(0, NUM_CHUNKS, body, 0)

    for d in range(GLEAD):
        wait_scatter(NUM_CHUNKS - GLEAD + d)

    plsc.subcore_barrier()

    half = RSLOTS * CHUNK // 2 * 2
    for h in range(ROWS_PER_TILE // half):
        rows = pl.ds(row0 + h * half, half)
        stage = rows_v.at[pl.ds(0, half)]
        pltpu.sync_copy(agg_sh.at[rows], stage)

        @pl.when(cid == 0)
        def _c0():
            pltpu.sync_copy(stage, out0_hbm.at[rows])

        @pl.when(cid == 1)
        def _c1():
            pltpu.sync_copy(stage, out1_hbm.at[rows])


_sc_agg = functools.partial(
    pl.kernel,
    out_type=(jax.ShapeDtypeStruct((NPAD, D), jnp.float32),
              jax.ShapeDtypeStruct((NPAD, D), jnp.float32)),
    mesh=plsc.VectorSubcoreMesh(core_axis_name="c", subcore_axis_name="s"),
    scratch_types=[
        pltpu.VMEM((ISLOTS, CHUNK), jnp.int32),
        pltpu.VMEM((ISLOTS, CHUNK), jnp.int32),
        pltpu.VMEM((RSLOTS * CHUNK, D), jnp.float32),
        pltpu.VMEM_SHARED((NPAD, D), jnp.float32),
        pltpu.SemaphoreType.DMA((RSLOTS,)),
        pltpu.SemaphoreType.DMA((RSLOTS,)),
        pltpu.SemaphoreType.DMA((ISLOTS,)),
        pltpu.SemaphoreType.DMA,
        pltpu.SemaphoreType.DMA((2,)),
        pltpu.SemaphoreType.DMA((2,)),
    ],
)(_sc_agg_body)


ROWS_BLK = 2000


def _mlp_body(x_ref, a0_ref, a1_ref, w1_ref, b1_ref, w2_ref, b2_ref, o_ref):
    h = x_ref[...] + a0_ref[...] + a1_ref[...]
    h = lax.dot_general(h, w1_ref[...], (((1,), (1,)), ((), ())),
                        preferred_element_type=jnp.float32) + b1_ref[...]
    h = jnp.maximum(h, 0.0)
    h = lax.dot_general(h, w2_ref[...], (((1,), (1,)), ((), ())),
                        preferred_element_type=jnp.float32) + b2_ref[...]
    o_ref[...] = jnp.maximum(h, 0.0)


def _mlp(x, a0, a1, W1, b1, W2, b2):
    grid = (N // ROWS_BLK,)
    row_spec = pl.BlockSpec((ROWS_BLK, D), lambda i: (i, 0))
    full_spec = pl.BlockSpec((D, D), lambda i: (0, 0))
    bias_spec = pl.BlockSpec((D,), lambda i: (0,))
    return pl.pallas_call(
        _mlp_body,
        grid=grid,
        in_specs=[row_spec, row_spec, row_spec,
                  full_spec, bias_spec, full_spec, bias_spec],
        out_specs=row_spec,
        out_shape=jax.ShapeDtypeStruct((N, D), jnp.float32),
    )(x, a0, a1, W1, b1, W2, b2)


def kernel(x, edge_index, W1, b1, W2, b2):
    src = edge_index[0]
    dst = edge_index[1]
    a0, a1 = _sc_agg(src, dst, x)
    return _mlp(x, a0, a1, W1, b1, W2, b2)
